# baseline (speedup 1.0000x reference)
import functools

import jax
import jax.numpy as jnp
from jax import lax
from jax.experimental import pallas as pl
from jax.experimental.pallas import tpu as pltpu
from jax.experimental.pallas import tpu_sc as plsc

N_CODES = 8192
CODE_DIM = 256
N_TOKENS = 8192
TM = 512


def _argmin_body(x_ref, wt_ref, wtb_ref, idx_ref):
    x = x_ref[...]
    wt = wt_ref[...]
    xsq = jnp.sum(x * x, axis=1, keepdims=True)
    wsq = jnp.sum(wt * wt, axis=0, keepdims=True)
    mm = jnp.dot(x.astype(jnp.bfloat16), wtb_ref[...],
                 preferred_element_type=jnp.float32)
    dist = xsq - 2.0 * mm + wsq
    idx = jnp.argmax(-dist, axis=1).astype(jnp.int32)
    idx_ref[...] = idx.reshape(1, 1, TM)


def _nearest_code(flat, wt):
    grid = N_TOKENS // TM
    idx_blocks = pl.pallas_call(
        _argmin_body,
        grid=(grid,),
        in_specs=[
            pl.BlockSpec((TM, CODE_DIM), lambda i: (i, 0)),
            pl.BlockSpec((CODE_DIM, N_CODES), lambda i: (0, 0)),
            pl.BlockSpec((CODE_DIM, N_CODES), lambda i: (0, 0)),
        ],
        out_specs=pl.BlockSpec((1, 1, TM), lambda i: (i, 0, 0)),
        out_shape=jax.ShapeDtypeStruct((grid, 1, TM), jnp.int32),
    )(flat, wt, wt.astype(jnp.bfloat16))
    return idx_blocks.reshape(-1)


_NC, _NS = 2, 16
_NW = _NC * _NS
_B_PER_W = N_TOKENS // _NW
_CH = 128


def _gather_body(table_hbm, idx_hbm, out_hbm,
                 idx_v0, idx_v1, rows_v0, rows_v1, sem0, sem1):
    wid = lax.axis_index("s") * _NC + lax.axis_index("c")
    base = wid * _B_PER_W
    pltpu.sync_copy(idx_hbm.at[pl.ds(base, _CH)], idx_v0)
    pltpu.sync_copy(idx_hbm.at[pl.ds(base + _CH, _CH)], idx_v1)
    cp0 = pltpu.async_copy(table_hbm.at[idx_v0], rows_v0, sem0)
    cp1 = pltpu.async_copy(table_hbm.at[idx_v1], rows_v1, sem1)
    cp0.wait()
    cp1.wait()
    pltpu.sync_copy(rows_v0, out_hbm.at[pl.ds(base, _CH)])
    pltpu.sync_copy(rows_v1, out_hbm.at[pl.ds(base + _CH, _CH)])


@jax.jit
def _sc_gather(table, idx):
    mesh = plsc.VectorSubcoreMesh(core_axis_name="c", subcore_axis_name="s")
    k = functools.partial(
        pl.kernel, mesh=mesh,
        out_type=jax.ShapeDtypeStruct((N_TOKENS, CODE_DIM), jnp.float32),
        scratch_types=[
            pltpu.VMEM((_CH,), jnp.int32),
            pltpu.VMEM((_CH,), jnp.int32),
            pltpu.VMEM((_CH, CODE_DIM), jnp.float32),
            pltpu.VMEM((_CH, CODE_DIM), jnp.float32),
            pltpu.SemaphoreType.DMA,
            pltpu.SemaphoreType.DMA,
        ],
    )(_gather_body)
    return k(table, idx)


def kernel(z, weight):
    b, c, h, w = z.shape
    flat = jnp.transpose(z, (0, 2, 3, 1)).reshape(-1, c)
    wt = weight.T
    idx = _nearest_code(flat, wt)
    q_flat = _sc_gather(weight, idx)
    quantized = q_flat.reshape(b, h, w, c).transpose(0, 3, 1, 2)
    encoding_indices = idx.reshape(b, h, w)
    return quantized, quantized, encoding_indices
